# baseline (speedup 1.0000x reference)
import functools

import jax
import jax.numpy as jnp
from jax import lax
from jax.experimental import pallas as pl
from jax.experimental.pallas import tpu as pltpu
from jax.experimental.pallas import tpu_sc as plsc


@functools.lru_cache(maxsize=None)
def _build_copy(seq_len: int, embed_dim: int, dtype_name: str):
    dtype = jnp.dtype(dtype_name)
    info = plsc.get_sparse_core_info()
    nc, ns = info.num_cores, info.num_subcores
    nw = nc * ns
    assert seq_len % nw == 0
    rows_per_w = seq_len // nw

    mesh = plsc.VectorSubcoreMesh(core_axis_name="c", subcore_axis_name="s")

    def body(table_hbm, out_hbm):
        wid = lax.axis_index("s") * nc + lax.axis_index("c")
        base = wid * rows_per_w
        pltpu.sync_copy(
            table_hbm.at[pl.ds(base, rows_per_w)],
            out_hbm.at[pl.ds(base, rows_per_w)],
        )

    return pl.kernel(
        body,
        out_type=jax.ShapeDtypeStruct((seq_len, embed_dim), dtype),
        mesh=mesh,
    )


def kernel(idx, table):
    seq_len = idx.shape[1]
    return _build_copy(seq_len, table.shape[1], table.dtype.name)(table)


# speedup vs baseline: 23.7819x; 23.7819x over previous
import functools

import jax
import jax.numpy as jnp
from jax import lax
from jax.experimental import pallas as pl
from jax.experimental.pallas import tpu as pltpu
from jax.experimental.pallas import tpu_sc as plsc


_CHUNK = 32
_NBUF = 3


@functools.lru_cache(maxsize=None)
def _build_copy(seq_len: int, embed_dim: int, dtype_name: str):
    dtype = jnp.dtype(dtype_name)
    info = plsc.get_sparse_core_info()
    nc, ns = info.num_cores, info.num_subcores
    nw = nc * ns
    assert seq_len % (nw * _CHUNK) == 0
    rows_per_w = seq_len // nw
    nchunks = rows_per_w // _CHUNK

    mesh = plsc.VectorSubcoreMesh(core_axis_name="c", subcore_axis_name="s")

    def body(table_hbm, out_hbm, *scratch):
        bufs = scratch[:_NBUF]
        in_sems = scratch[_NBUF:2 * _NBUF]
        out_sems = scratch[2 * _NBUF:3 * _NBUF]
        wid = lax.axis_index("s") * nc + lax.axis_index("c")
        base = wid * rows_per_w

        in_d = [None] * nchunks
        out_d = [None] * nchunks
        for i in range(nchunks):
            b = i % _NBUF
            if i >= _NBUF:
                out_d[i - _NBUF].wait()
            in_d[i] = pltpu.async_copy(
                table_hbm.at[pl.ds(base + i * _CHUNK, _CHUNK)], bufs[b],
                in_sems[b])
            if i >= 1:
                j = i - 1
                in_d[j].wait()
                out_d[j] = pltpu.async_copy(
                    bufs[j % _NBUF],
                    out_hbm.at[pl.ds(base + j * _CHUNK, _CHUNK)],
                    out_sems[j % _NBUF])
        j = nchunks - 1
        in_d[j].wait()
        out_d[j] = pltpu.async_copy(
            bufs[j % _NBUF], out_hbm.at[pl.ds(base + j * _CHUNK, _CHUNK)],
            out_sems[j % _NBUF])
        for j in range(max(0, nchunks - _NBUF), nchunks):
            out_d[j].wait()

    return pl.kernel(
        body,
        out_type=jax.ShapeDtypeStruct((seq_len, embed_dim), dtype),
        mesh=mesh,
        scratch_types=(
            [pltpu.VMEM((_CHUNK, embed_dim), dtype) for _ in range(_NBUF)]
            + [pltpu.SemaphoreType.DMA for _ in range(2 * _NBUF)]
        ),
    )


def kernel(idx, table):
    seq_len = idx.shape[1]
    return _build_copy(seq_len, table.shape[1], table.dtype.name)(table)


# speedup vs baseline: 24.5794x; 1.0335x over previous
import functools

import jax
import jax.numpy as jnp
from jax import lax
from jax.experimental import pallas as pl
from jax.experimental.pallas import tpu as pltpu
from jax.experimental.pallas import tpu_sc as plsc


_CHUNK = 16
_NBUF = 7
_LAG = 3


@functools.lru_cache(maxsize=None)
def _build_copy(seq_len: int, embed_dim: int, dtype_name: str):
    dtype = jnp.dtype(dtype_name)
    info = plsc.get_sparse_core_info()
    nc, ns = info.num_cores, info.num_subcores
    nw = nc * ns
    assert seq_len % (nw * _CHUNK) == 0
    rows_per_w = seq_len // nw
    nchunks = rows_per_w // _CHUNK

    mesh = plsc.VectorSubcoreMesh(core_axis_name="c", subcore_axis_name="s")

    def body(table_hbm, out_hbm, *scratch):
        bufs = scratch[:_NBUF]
        in_sems = scratch[_NBUF:2 * _NBUF]
        out_sems = scratch[2 * _NBUF:3 * _NBUF]
        wid = lax.axis_index("s") * nc + lax.axis_index("c")
        base = wid * rows_per_w

        in_d = [None] * nchunks
        out_d = [None] * nchunks
        for i in range(nchunks + _LAG):
            if i < nchunks:
                b = i % _NBUF
                if i >= _NBUF:
                    out_d[i - _NBUF].wait()
                in_d[i] = pltpu.async_copy(
                    table_hbm.at[pl.ds(base + i * _CHUNK, _CHUNK)], bufs[b],
                    in_sems[b])
            if i >= _LAG:
                j = i - _LAG
                in_d[j].wait()
                out_d[j] = pltpu.async_copy(
                    bufs[j % _NBUF],
                    out_hbm.at[pl.ds(base + j * _CHUNK, _CHUNK)],
                    out_sems[j % _NBUF])
        for j in range(max(0, nchunks - _NBUF), nchunks):
            out_d[j].wait()

    return pl.kernel(
        body,
        out_type=jax.ShapeDtypeStruct((seq_len, embed_dim), dtype),
        mesh=mesh,
        scratch_types=(
            [pltpu.VMEM((_CHUNK, embed_dim), dtype) for _ in range(_NBUF)]
            + [pltpu.SemaphoreType.DMA for _ in range(2 * _NBUF)]
        ),
    )


def kernel(idx, table):
    seq_len = idx.shape[1]
    return _build_copy(seq_len, table.shape[1], table.dtype.name)(table)
